# Initial kernel scaffold; baseline (speedup 1.0000x reference)
import jax
import jax.numpy as jnp
from jax.experimental import pallas as pl


def kernel(x, edge_index, W1, b1, W2, b2, W3, b3, ln_g, ln_b):
    raise NotImplementedError("write your pallas kernel here")


# speedup vs baseline: 14.1390x; 14.1390x over previous
import functools

import jax
import jax.numpy as jnp
from jax import lax
from jax.experimental import pallas as pl
from jax.experimental.pallas import tpu as pltpu
from jax.experimental.pallas import tpu_sc as plsc

_N = 10000
_NT = 16
_NC = 2
_B = 128
_D = 128
_PAD_ROWS = 16
_ACC_ROWS = 10240
_OCH = 632
_IC = 8
_NP = _NT * _OCH
_BN = 1000


def _make_prop(nbt, split):
  mesh = plsc.VectorSubcoreMesh(core_axis_name="c", subcore_axis_name="s")
  zch = _ACC_ROWS // _NT
  och = _OCH

  @functools.partial(
      pl.kernel,
      out_type=jax.ShapeDtypeStruct((_NC * _NP, _D), jnp.float32),
      mesh=mesh,
      scratch_types=[
          pltpu.VMEM((_IC, _B), jnp.int32),
          pltpu.VMEM((_IC, _B), jnp.int32),
          pltpu.VMEM((_B,), jnp.int32),
          pltpu.VMEM((_B, _D), jnp.float32),
          pltpu.VMEM_SHARED((_ACC_ROWS, _D), jnp.float32),
          pltpu.SemaphoreType.DMA,
      ],
  )
  def prop(xs_hbm, row_hbm, col_hbm, out_hbm, row_v, col_v, idx_v, rows_v,
           acc, sem):
    c = lax.axis_index("c")
    s = lax.axis_index("s")
    zero16 = jnp.zeros((16,), jnp.float32)

    def _zb(i, carry):
      rows_v[i // (_D // 16), pl.ds((i % (_D // 16)) * 16, 16)] = zero16
      return carry
    lax.fori_loop(0, _B * (_D // 16), _zb, 0)
    zbase = s * zch
    for k in range(zch // _B):
      pltpu.sync_copy(rows_v, acc.at[pl.ds(zbase + k * _B, _B)])
    plsc.subcore_barrier()

    ibase = s * nbt if split else (c * _NT + s) * nbt
    cn16 = jnp.zeros((16,), jnp.int32) + c * _N

    def _chunk(q, carry):
      cbase = ibase + q * _IC
      pltpu.sync_copy(row_hbm.at[pl.ds(cbase, _IC)], row_v)
      pltpu.sync_copy(col_hbm.at[pl.ds(cbase, _IC)], col_v)
      for j in range(_IC):
        if split:
          for i in range(_B // 16):
            idx_v[pl.ds(i * 16, 16)] = row_v[j, pl.ds(i * 16, 16)] + cn16
          pltpu.async_copy(xs_hbm.at[idx_v], rows_v, sem).wait()
        else:
          pltpu.async_copy(xs_hbm.at[row_v.at[j]], rows_v, sem).wait()
        pltpu.sync_copy(rows_v, acc.at[col_v.at[j]], add=True)
      return carry

    lax.fori_loop(0, nbt // _IC, _chunk, 0)
    plsc.subcore_barrier()

    src0 = _PAD_ROWS + s * och
    dst0 = c * _NP + s * och
    for k in range(och // _B):
      pltpu.sync_copy(acc.at[pl.ds(src0 + k * _B, _B)], rows_v)
      pltpu.sync_copy(rows_v, out_hbm.at[pl.ds(dst0 + k * _B, _B)])
    orem = och % _B
    if orem:
      o0 = (och // _B) * _B
      pltpu.sync_copy(acc.at[pl.ds(src0 + o0, orem)],
                      rows_v.at[pl.ds(0, orem)])
      pltpu.sync_copy(rows_v.at[pl.ds(0, orem)],
                      out_hbm.at[pl.ds(dst0 + o0, orem)])

  return prop


def _row_spec(d):
  return pl.BlockSpec((_BN, d), lambda i: (i, 0))


def _pair_spec(d):
  return pl.BlockSpec((2, _BN, d), lambda i: (0, i, 0))


def _full_spec(r, d):
  return pl.BlockSpec((r, d), lambda i: (0, 0))


def _tca(deg_ref, x_ref, xs_ref, dis_ref):
  dis = lax.rsqrt(deg_ref[...] + 1.0)
  xs_ref[...] = x_ref[...] * dis
  dis_ref[...] = dis


def _tcb(s1_ref, xs_ref, dis_ref, w_ref, b_ref, out_ref):
  s1 = s1_ref[...]
  dis = dis_ref[...]
  t = (s1[0] + s1[1] + xs_ref[...]) * dis
  pre = jnp.dot(t, w_ref[...], preferred_element_type=jnp.float32) + b_ref[...]
  h = jnp.maximum(pre, 0.0) * dis
  out_ref[0, :, :] = h[:, :128]
  out_ref[1, :, :] = h[:, 128:]


def _tcc(s2_ref, h1_ref, dis_ref, w2_ref, b2_ref, w3_ref, out_ref):
  s2 = s2_ref[...]
  h1 = h1_ref[...]
  dis = dis_ref[...]
  t = jnp.concatenate([s2[0] + h1[0], s2[1] + h1[1]], axis=1) * dis
  h2 = jnp.maximum(
      jnp.dot(t, w2_ref[...], preferred_element_type=jnp.float32)
      + b2_ref[...], 0.0)
  out_ref[...] = jnp.dot(h2, w3_ref[...],
                         preferred_element_type=jnp.float32) * dis


def _tcd(s3_ref, vs_ref, dis_ref, b3_ref, g_ref, lb_ref, out_ref):
  s3 = s3_ref[...]
  h = (s3[0] + s3[1] + vs_ref[...]) * dis_ref[...] + b3_ref[...]
  m = jnp.mean(h, axis=1, keepdims=True)
  v = jnp.mean((h - m) * (h - m), axis=1, keepdims=True)
  out_ref[...] = (h - m) * lax.rsqrt(v + 1e-5) * g_ref[...] + lb_ref[...]


def kernel(x, edge_index, W1, b1, W2, b2, W3, b3, ln_g, ln_b):
  n = x.shape[0]
  e = edge_index.shape[1]
  grid = (n // _BN,)
  epad = -(-e // (_NC * _NT * _B * 8)) * (_NC * _NT * _B * 8)
  nbt_half = epad // (_NC * _NT * _B)
  nbt_full = epad // (_NT * _B)
  p = epad - e

  pad_row = (jnp.arange(p, dtype=jnp.int32) * 37) % n
  pad_col = jnp.arange(p, dtype=jnp.int32) % _PAD_ROWS
  rowp = jnp.concatenate([edge_index[0], pad_row]).reshape(-1, _B)
  colp = jnp.concatenate([edge_index[1] + _PAD_ROWS, pad_col]).reshape(-1, _B)

  prop_part = _make_prop(nbt_half, split=False)
  prop_splt = _make_prop(nbt_full, split=True)

  degp = prop_part(jnp.ones((n, _D), jnp.float32), rowp, colp)
  deg = degp[0:n, 0:1] + degp[_NP:_NP + n, 0:1]

  xs, dis = pl.pallas_call(
      _tca,
      grid=grid,
      in_specs=[_row_spec(1), _row_spec(128)],
      out_specs=[_row_spec(128), _row_spec(1)],
      out_shape=[
          jax.ShapeDtypeStruct((n, 128), jnp.float32),
          jax.ShapeDtypeStruct((n, 1), jnp.float32),
      ],
  )(deg, x)

  s1 = prop_part(xs, rowp, colp).reshape(2, _NP, _D)

  h1 = pl.pallas_call(
      _tcb,
      grid=grid,
      in_specs=[_pair_spec(128), _row_spec(128), _row_spec(1),
                _full_spec(128, 256), _full_spec(1, 256)],
      out_specs=_pair_spec(128),
      out_shape=jax.ShapeDtypeStruct((2, n, 128), jnp.float32),
  )(s1, xs, dis, W1, b1.reshape(1, -1))

  s2 = prop_splt(h1.reshape(2 * n, 128), rowp, colp).reshape(2, _NP, _D)

  vs = pl.pallas_call(
      _tcc,
      grid=grid,
      in_specs=[_pair_spec(128), _pair_spec(128), _row_spec(1),
                _full_spec(256, 256), _full_spec(1, 256),
                _full_spec(256, 128)],
      out_specs=_row_spec(128),
      out_shape=jax.ShapeDtypeStruct((n, 128), jnp.float32),
  )(s2, h1, dis, W2, b2.reshape(1, -1), W3)

  s3 = prop_part(vs, rowp, colp).reshape(2, _NP, _D)

  out = pl.pallas_call(
      _tcd,
      grid=grid,
      in_specs=[_pair_spec(128), _row_spec(128), _row_spec(1),
                _full_spec(1, 128), _full_spec(1, 128), _full_spec(1, 128)],
      out_specs=_row_spec(128),
      out_shape=jax.ShapeDtypeStruct((n, 128), jnp.float32),
  )(s3, vs, dis, b3.reshape(1, -1), ln_g.reshape(1, -1), ln_b.reshape(1, -1))

  return out


# speedup vs baseline: 20.3261x; 1.4376x over previous
import functools

import jax
import jax.numpy as jnp
from jax import lax
from jax.experimental import pallas as pl
from jax.experimental.pallas import tpu as pltpu
from jax.experimental.pallas import tpu_sc as plsc

_N = 10000
_NT = 16
_NC = 2
_B = 128
_D = 128
_PAD_ROWS = 16
_ACC_ROWS = 10240
_OCH = 632
_IC = 8
_NP = _NT * _OCH
_BN = 1000


def _make_prop(nbt, mode):
  mesh = plsc.VectorSubcoreMesh(core_axis_name="c", subcore_axis_name="s")
  zch = _ACC_ROWS // _NT
  och = _OCH
  split = mode == "split"
  gather = mode != "ones"

  @functools.partial(
      pl.kernel,
      out_type=jax.ShapeDtypeStruct((_NC * _NP, _D), jnp.float32),
      mesh=mesh,
      scratch_types=[
          pltpu.VMEM((_IC, _B), jnp.int32),
          pltpu.VMEM((_IC, _B), jnp.int32),
          pltpu.VMEM((2, _B), jnp.int32),
          pltpu.VMEM((2, _B, _D), jnp.float32),
          pltpu.VMEM_SHARED((_ACC_ROWS, _D), jnp.float32),
          pltpu.SemaphoreType.DMA,
          pltpu.SemaphoreType.DMA,
      ],
  )
  def prop(xs_hbm, row_hbm, col_hbm, out_hbm, row_v, col_v, idx_v, rows_v,
           acc, sem0, sem1):
    c = lax.axis_index("c")
    s = lax.axis_index("s")
    sems = (sem0, sem1)
    fill16 = jnp.full((16,), 0.0 if gather else 1.0, jnp.float32)

    def _fill(i, carry):
      rows_v[(i // (_D // 16)) % 2, i // (2 * (_D // 16)),
             pl.ds((i % (_D // 16)) * 16, 16)] = fill16
      return carry
    lax.fori_loop(0, 2 * _B * (_D // 16), _fill, 0)
    zbase = s * zch
    if gather:
      for k in range(zch // _B):
        pltpu.sync_copy(rows_v.at[0], acc.at[pl.ds(zbase + k * _B, _B)])
    else:
      def _z0(i, carry):
        rows_v[0, i // (_D // 16),
               pl.ds((i % (_D // 16)) * 16, 16)] = jnp.zeros((16,),
                                                             jnp.float32)
        return carry
      lax.fori_loop(0, _B * (_D // 16), _z0, 0)
      for k in range(zch // _B):
        pltpu.sync_copy(rows_v.at[0], acc.at[pl.ds(zbase + k * _B, _B)])
      def _r0(i, carry):
        rows_v[0, i // (_D // 16),
               pl.ds((i % (_D // 16)) * 16, 16)] = jnp.ones((16,),
                                                            jnp.float32)
        return carry
      lax.fori_loop(0, _B * (_D // 16), _r0, 0)
    plsc.subcore_barrier()

    ibase = s * nbt if split else (c * _NT + s) * nbt
    cn16 = jnp.zeros((16,), jnp.int32) + c * _N

    def _set_idx(j, p):
      for i in range(_B // 16):
        idx_v[p, pl.ds(i * 16, 16)] = row_v[j, pl.ds(i * 16, 16)] + cn16

    def _start(j, p):
      if split:
        _set_idx(j, p)
        return pltpu.async_copy(xs_hbm.at[idx_v.at[p]], rows_v.at[p],
                                sems[p])
      return pltpu.async_copy(xs_hbm.at[row_v.at[j]], rows_v.at[p], sems[p])

    def _chunk(q, carry):
      cbase = ibase + q * _IC
      pltpu.sync_copy(row_hbm.at[pl.ds(cbase, _IC)], row_v)
      pltpu.sync_copy(col_hbm.at[pl.ds(cbase, _IC)], col_v)
      if gather:
        h = _start(0, 0)
        for j in range(_IC):
          hn = _start(j + 1, (j + 1) % 2) if j + 1 < _IC else None
          h.wait()
          pltpu.sync_copy(rows_v.at[j % 2], acc.at[col_v.at[j]], add=True)
          h = hn
      else:
        for j in range(_IC):
          pltpu.sync_copy(rows_v.at[0], acc.at[col_v.at[j]], add=True)
      return carry

    lax.fori_loop(0, nbt // _IC, _chunk, 0)
    plsc.subcore_barrier()

    src0 = _PAD_ROWS + s * och
    dst0 = c * _NP + s * och
    for k in range(och // _B):
      pltpu.sync_copy(acc.at[pl.ds(src0 + k * _B, _B)], rows_v.at[0])
      pltpu.sync_copy(rows_v.at[0], out_hbm.at[pl.ds(dst0 + k * _B, _B)])
    orem = och % _B
    if orem:
      o0 = (och // _B) * _B
      pltpu.sync_copy(acc.at[pl.ds(src0 + o0, orem)],
                      rows_v.at[0].at[pl.ds(0, orem)])
      pltpu.sync_copy(rows_v.at[0].at[pl.ds(0, orem)],
                      out_hbm.at[pl.ds(dst0 + o0, orem)])

  return prop


def _row_spec(d):
  return pl.BlockSpec((_BN, d), lambda i: (i, 0))


def _pair_spec(d):
  return pl.BlockSpec((2, _BN, d), lambda i: (0, i, 0))


def _full_spec(r, d):
  return pl.BlockSpec((r, d), lambda i: (0, 0))


def _tca(deg_ref, x_ref, xs_ref, dis_ref):
  dis = lax.rsqrt(deg_ref[...] + 1.0)
  xs_ref[...] = x_ref[...] * dis
  dis_ref[...] = dis


def _tcb(s1_ref, xs_ref, dis_ref, w_ref, b_ref, out_ref):
  s1 = s1_ref[...]
  dis = dis_ref[...]
  t = (s1[0] + s1[1] + xs_ref[...]) * dis
  pre = jnp.dot(t, w_ref[...], preferred_element_type=jnp.float32) + b_ref[...]
  h = jnp.maximum(pre, 0.0) * dis
  out_ref[0, :, :] = h[:, :128]
  out_ref[1, :, :] = h[:, 128:]


def _tcc(s2_ref, h1_ref, dis_ref, w2_ref, b2_ref, w3_ref, out_ref):
  s2 = s2_ref[...]
  h1 = h1_ref[...]
  dis = dis_ref[...]
  t = jnp.concatenate([s2[0] + h1[0], s2[1] + h1[1]], axis=1) * dis
  h2 = jnp.maximum(
      jnp.dot(t, w2_ref[...], preferred_element_type=jnp.float32)
      + b2_ref[...], 0.0)
  out_ref[...] = jnp.dot(h2, w3_ref[...],
                         preferred_element_type=jnp.float32) * dis


def _tcd(s3_ref, vs_ref, dis_ref, b3_ref, g_ref, lb_ref, out_ref):
  s3 = s3_ref[...]
  h = (s3[0] + s3[1] + vs_ref[...]) * dis_ref[...] + b3_ref[...]
  m = jnp.mean(h, axis=1, keepdims=True)
  v = jnp.mean((h - m) * (h - m), axis=1, keepdims=True)
  out_ref[...] = (h - m) * lax.rsqrt(v + 1e-5) * g_ref[...] + lb_ref[...]


def kernel(x, edge_index, W1, b1, W2, b2, W3, b3, ln_g, ln_b):
  n = x.shape[0]
  e = edge_index.shape[1]
  grid = (n // _BN,)
  epad = -(-e // (_NC * _NT * _B * 8)) * (_NC * _NT * _B * 8)
  nbt_half = epad // (_NC * _NT * _B)
  nbt_full = epad // (_NT * _B)
  p = epad - e

  pad_row = (jnp.arange(p, dtype=jnp.int32) * 37) % n
  pad_col = jnp.arange(p, dtype=jnp.int32) % _PAD_ROWS
  rowp = jnp.concatenate([edge_index[0], pad_row]).reshape(-1, _B)
  colp = jnp.concatenate([edge_index[1] + _PAD_ROWS, pad_col]).reshape(-1, _B)

  prop_part = _make_prop(nbt_half, "part")
  prop_splt = _make_prop(nbt_full, "split")
  prop_ones = _make_prop(nbt_half, "ones")

  degp = prop_ones(jnp.ones((8, _D), jnp.float32), rowp, colp)
  deg = degp[0:n, 0:1] + degp[_NP:_NP + n, 0:1]

  xs, dis = pl.pallas_call(
      _tca,
      grid=grid,
      in_specs=[_row_spec(1), _row_spec(128)],
      out_specs=[_row_spec(128), _row_spec(1)],
      out_shape=[
          jax.ShapeDtypeStruct((n, 128), jnp.float32),
          jax.ShapeDtypeStruct((n, 1), jnp.float32),
      ],
  )(deg, x)

  s1 = prop_part(xs, rowp, colp).reshape(2, _NP, _D)

  h1 = pl.pallas_call(
      _tcb,
      grid=grid,
      in_specs=[_pair_spec(128), _row_spec(128), _row_spec(1),
                _full_spec(128, 256), _full_spec(1, 256)],
      out_specs=_pair_spec(128),
      out_shape=jax.ShapeDtypeStruct((2, n, 128), jnp.float32),
  )(s1, xs, dis, W1, b1.reshape(1, -1))

  s2 = prop_splt(h1.reshape(2 * n, 128), rowp, colp).reshape(2, _NP, _D)

  vs = pl.pallas_call(
      _tcc,
      grid=grid,
      in_specs=[_pair_spec(128), _pair_spec(128), _row_spec(1),
                _full_spec(256, 256), _full_spec(1, 256),
                _full_spec(256, 128)],
      out_specs=_row_spec(128),
      out_shape=jax.ShapeDtypeStruct((n, 128), jnp.float32),
  )(s2, h1, dis, W2, b2.reshape(1, -1), W3)

  s3 = prop_part(vs, rowp, colp).reshape(2, _NP, _D)

  out = pl.pallas_call(
      _tcd,
      grid=grid,
      in_specs=[_pair_spec(128), _row_spec(128), _row_spec(1),
                _full_spec(1, 128), _full_spec(1, 128), _full_spec(1, 128)],
      out_specs=_row_spec(128),
      out_shape=jax.ShapeDtypeStruct((n, 128), jnp.float32),
  )(s3, vs, dis, b3.reshape(1, -1), ln_g.reshape(1, -1), ln_b.reshape(1, -1))

  return out


# speedup vs baseline: 23.2123x; 1.1420x over previous
import functools

import jax
import jax.numpy as jnp
from jax import lax
from jax.experimental import pallas as pl
from jax.experimental.pallas import tpu as pltpu
from jax.experimental.pallas import tpu_sc as plsc

_N = 10000
_NT = 16
_NC = 2
_B = 128
_D = 128
_PAD_ROWS = 16
_ACC_ROWS = 10240
_OCH = 632
_IC = 8
_NP = _NT * _OCH
_BN = 1000


def _make_prop(nbt, mode):
  mesh = plsc.VectorSubcoreMesh(core_axis_name="c", subcore_axis_name="s")
  zch = _ACC_ROWS // _NT
  och = _OCH
  split = mode == "split"
  gather = mode != "ones"

  @functools.partial(
      pl.kernel,
      out_type=jax.ShapeDtypeStruct((_NC * _NP, _D), jnp.float32),
      mesh=mesh,
      scratch_types=[
          pltpu.VMEM((2, _IC, _B), jnp.int32),
          pltpu.VMEM((2, _IC, _B), jnp.int32),
          pltpu.VMEM((2, _B), jnp.int32),
          pltpu.VMEM((2, _B, _D), jnp.float32),
          pltpu.VMEM_SHARED((_ACC_ROWS, _D), jnp.float32),
          pltpu.SemaphoreType.DMA,
          pltpu.SemaphoreType.DMA,
          pltpu.SemaphoreType.DMA,
          pltpu.SemaphoreType.DMA,
          pltpu.SemaphoreType.DMA,
      ],
  )
  def prop(xs_hbm, row_hbm, col_hbm, out_hbm, row_v, col_v, idx_v, rows_v,
           acc, sem0, sem1, sem2, sem3, sem4):
    c = lax.axis_index("c")
    s = lax.axis_index("s")
    sems = (sem0, sem1)
    ssems = (sem2, sem3)
    isem = sem4

    b1val = jnp.full((16,), 0.0 if gather else 1.0, jnp.float32)
    zero16 = jnp.zeros((16,), jnp.float32)

    def _fill(r, carry):
      for i in range(_D // 16):
        rows_v[0, r, pl.ds(i * 16, 16)] = zero16
        rows_v[1, r, pl.ds(i * 16, 16)] = b1val
      return carry
    lax.fori_loop(0, _B, _fill, 0)
    zbase = s * zch
    for k in range(zch // _B):
      pltpu.sync_copy(rows_v.at[0], acc.at[pl.ds(zbase + k * _B, _B)])
    plsc.subcore_barrier()

    ibase = s * nbt if split else (c * _NT + s) * nbt
    cn16 = jnp.zeros((16,), jnp.int32) + c * _N
    nq = nbt // _IC

    def _set_idx(qb, j, p):
      for i in range(_B // 16):
        idx_v[p, pl.ds(i * 16, 16)] = row_v[qb, j, pl.ds(i * 16, 16)] + cn16

    def _start(qb, j, p):
      if split:
        _set_idx(qb, j, p)
        return pltpu.async_copy(xs_hbm.at[idx_v.at[p]], rows_v.at[p],
                                sems[p])
      return pltpu.async_copy(xs_hbm.at[row_v.at[qb].at[j]], rows_v.at[p],
                              sems[p])

    pltpu.sync_copy(row_hbm.at[pl.ds(ibase, _IC)], row_v.at[0])
    pltpu.sync_copy(col_hbm.at[pl.ds(ibase, _IC)], col_v.at[0])

    def _pair(q2, carry):
      for qb in range(2):
        q = q2 * 2 + qb
        cbase = ibase + q * _IC

        def _absorb():
          pltpu.make_async_copy(row_hbm.at[pl.ds(cbase, _IC)],
                                row_v.at[qb], isem).wait()
          pltpu.make_async_copy(col_hbm.at[pl.ds(cbase, _IC)],
                                col_v.at[qb], isem).wait()
        if qb == 0:
          @pl.when(q2 > 0)
          def _():
            _absorb()
        else:
          _absorb()

        @pl.when(q + 1 < nq)
        def _():
          pltpu.async_copy(row_hbm.at[pl.ds(cbase + _IC, _IC)],
                           row_v.at[1 - qb], isem)
          pltpu.async_copy(col_hbm.at[pl.ds(cbase + _IC, _IC)],
                           col_v.at[1 - qb], isem)

        cv = col_v.at[qb]
        if gather:
          h = _start(qb, 0, 0)
          for j in range(_IC):
            hn = _start(qb, j + 1, (j + 1) % 2) if j + 1 < _IC else None
            h.wait()
            pltpu.sync_copy(rows_v.at[j % 2], acc.at[cv.at[j]], add=True)
            h = hn
        else:
          for j in range(_IC):
            pltpu.sync_copy(rows_v.at[1], acc.at[cv.at[j]], add=True)
      return carry

    lax.fori_loop(0, nq // 2, _pair, 0)
    plsc.subcore_barrier()

    src0 = _PAD_ROWS + s * och
    dst0 = c * _NP + s * och
    for k in range(och // _B):
      pltpu.sync_copy(acc.at[pl.ds(src0 + k * _B, _B)], rows_v.at[0])
      pltpu.sync_copy(rows_v.at[0], out_hbm.at[pl.ds(dst0 + k * _B, _B)])
    orem = och % _B
    if orem:
      o0 = (och // _B) * _B
      pltpu.sync_copy(acc.at[pl.ds(src0 + o0, orem)],
                      rows_v.at[0].at[pl.ds(0, orem)])
      pltpu.sync_copy(rows_v.at[0].at[pl.ds(0, orem)],
                      out_hbm.at[pl.ds(dst0 + o0, orem)])

  return prop


def _row_spec(d):
  return pl.BlockSpec((_BN, d), lambda i: (i, 0))


def _pair_spec(d):
  return pl.BlockSpec((2, _BN, d), lambda i: (0, i, 0))


def _full_spec(r, d):
  return pl.BlockSpec((r, d), lambda i: (0, 0))


def _tca(deg_ref, x_ref, xs_ref, dis_ref):
  dis = lax.rsqrt(deg_ref[...] + 1.0)
  xs_ref[...] = x_ref[...] * dis
  dis_ref[...] = dis


def _tcb(s1_ref, xs_ref, dis_ref, w_ref, b_ref, out_ref):
  s1 = s1_ref[...]
  dis = dis_ref[...]
  t = (s1[0] + s1[1] + xs_ref[...]) * dis
  pre = jnp.dot(t, w_ref[...], preferred_element_type=jnp.float32) + b_ref[...]
  h = jnp.maximum(pre, 0.0) * dis
  out_ref[0, :, :] = h[:, :128]
  out_ref[1, :, :] = h[:, 128:]


def _tcc(s2_ref, h1_ref, dis_ref, w2_ref, b2_ref, w3_ref, out_ref):
  s2 = s2_ref[...]
  h1 = h1_ref[...]
  dis = dis_ref[...]
  t = jnp.concatenate([s2[0] + h1[0], s2[1] + h1[1]], axis=1) * dis
  h2 = jnp.maximum(
      jnp.dot(t, w2_ref[...], preferred_element_type=jnp.float32)
      + b2_ref[...], 0.0)
  out_ref[...] = jnp.dot(h2, w3_ref[...],
                         preferred_element_type=jnp.float32) * dis


def _tcd(s3_ref, vs_ref, dis_ref, b3_ref, g_ref, lb_ref, out_ref):
  s3 = s3_ref[...]
  h = (s3[0] + s3[1] + vs_ref[...]) * dis_ref[...] + b3_ref[...]
  m = jnp.mean(h, axis=1, keepdims=True)
  v = jnp.mean((h - m) * (h - m), axis=1, keepdims=True)
  out_ref[...] = (h - m) * lax.rsqrt(v + 1e-5) * g_ref[...] + lb_ref[...]


def kernel(x, edge_index, W1, b1, W2, b2, W3, b3, ln_g, ln_b):
  n = x.shape[0]
  e = edge_index.shape[1]
  grid = (n // _BN,)
  epad = -(-e // (_NC * _NT * _B * 8)) * (_NC * _NT * _B * 8)
  nbt_half = epad // (_NC * _NT * _B)
  nbt_full = epad // (_NT * _B)
  p = epad - e

  pad_row = (jnp.arange(p, dtype=jnp.int32) * 37) % n
  pad_col = jnp.arange(p, dtype=jnp.int32) % _PAD_ROWS
  rowp = jnp.concatenate([edge_index[0], pad_row]).reshape(-1, _B)
  colp = jnp.concatenate([edge_index[1] + _PAD_ROWS, pad_col]).reshape(-1, _B)

  prop_part = _make_prop(nbt_half, "part")
  prop_splt = _make_prop(nbt_full, "split")
  prop_ones = _make_prop(nbt_half, "ones")

  degp = prop_ones(jnp.ones((8, _D), jnp.float32), rowp, colp)
  deg = degp[0:n, 0:1] + degp[_NP:_NP + n, 0:1]

  xs, dis = pl.pallas_call(
      _tca,
      grid=grid,
      in_specs=[_row_spec(1), _row_spec(128)],
      out_specs=[_row_spec(128), _row_spec(1)],
      out_shape=[
          jax.ShapeDtypeStruct((n, 128), jnp.float32),
          jax.ShapeDtypeStruct((n, 1), jnp.float32),
      ],
  )(deg, x)

  s1 = prop_part(xs, rowp, colp).reshape(2, _NP, _D)

  h1 = pl.pallas_call(
      _tcb,
      grid=grid,
      in_specs=[_pair_spec(128), _row_spec(128), _row_spec(1),
                _full_spec(128, 256), _full_spec(1, 256)],
      out_specs=_pair_spec(128),
      out_shape=jax.ShapeDtypeStruct((2, n, 128), jnp.float32),
  )(s1, xs, dis, W1, b1.reshape(1, -1))

  s2 = prop_splt(h1.reshape(2 * n, 128), rowp, colp).reshape(2, _NP, _D)

  vs = pl.pallas_call(
      _tcc,
      grid=grid,
      in_specs=[_pair_spec(128), _pair_spec(128), _row_spec(1),
                _full_spec(256, 256), _full_spec(1, 256),
                _full_spec(256, 128)],
      out_specs=_row_spec(128),
      out_shape=jax.ShapeDtypeStruct((n, 128), jnp.float32),
  )(s2, h1, dis, W2, b2.reshape(1, -1), W3)

  s3 = prop_part(vs, rowp, colp).reshape(2, _NP, _D)

  out = pl.pallas_call(
      _tcd,
      grid=grid,
      in_specs=[_pair_spec(128), _row_spec(128), _row_spec(1),
                _full_spec(1, 128), _full_spec(1, 128), _full_spec(1, 128)],
      out_specs=_row_spec(128),
      out_shape=jax.ShapeDtypeStruct((n, 128), jnp.float32),
  )(s3, vs, dis, b3.reshape(1, -1), ln_g.reshape(1, -1), ln_b.reshape(1, -1))

  return out


# speedup vs baseline: 23.4312x; 1.0094x over previous
import functools

import jax
import jax.numpy as jnp
from jax import lax
from jax.experimental import pallas as pl
from jax.experimental.pallas import tpu as pltpu
from jax.experimental.pallas import tpu_sc as plsc

_N = 10000
_NT = 16
_NC = 2
_B = 128
_D = 128
_PAD_ROWS = 16
_ACC_ROWS = 10240
_OCH = 632
_IC = 8
_NP = _NT * _OCH
_BN = 1000


def _make_prop(nbt, mode):
  mesh = plsc.VectorSubcoreMesh(core_axis_name="c", subcore_axis_name="s")
  zch = _ACC_ROWS // _NT
  och = _OCH
  split = mode == "split"
  gather = mode != "ones"

  @functools.partial(
      pl.kernel,
      out_type=jax.ShapeDtypeStruct((_NC * _NP, _D), jnp.float32),
      mesh=mesh,
      scratch_types=[
          pltpu.VMEM((2, _IC, _B), jnp.int32),
          pltpu.VMEM((2, _IC, _B), jnp.int32),
          pltpu.VMEM((2, _B), jnp.int32),
          pltpu.VMEM((2, _B, _D), jnp.float32),
          pltpu.VMEM_SHARED((_ACC_ROWS, _D), jnp.float32),
          pltpu.SemaphoreType.DMA,
          pltpu.SemaphoreType.DMA,
          pltpu.SemaphoreType.DMA,
          pltpu.SemaphoreType.DMA,
          pltpu.SemaphoreType.DMA,
      ],
  )
  def prop(xs_hbm, row_hbm, col_hbm, out_hbm, row_v, col_v, idx_v, rows_v,
           acc, sem0, sem1, sem2, sem3, sem4):
    c = lax.axis_index("c")
    s = lax.axis_index("s")
    sems = (sem0, sem1)
    ssems = (sem2, sem3)
    isem = sem4

    b1val = jnp.full((16,), 0.0 if gather else 1.0, jnp.float32)
    zero16 = jnp.zeros((16,), jnp.float32)

    def _fill(r, carry):
      for i in range(_D // 16):
        rows_v[0, r, pl.ds(i * 16, 16)] = zero16
        rows_v[1, r, pl.ds(i * 16, 16)] = b1val
      return carry
    lax.fori_loop(0, _B, _fill, 0)
    zbase = s * zch
    hz = [pltpu.async_copy(rows_v.at[0], acc.at[pl.ds(zbase + k * _B, _B)],
                           sem2) for k in range(zch // _B)]
    for h in hz:
      h.wait()
    plsc.subcore_barrier()

    ibase = s * nbt if split else (c * _NT + s) * nbt
    cn16 = jnp.zeros((16,), jnp.int32) + c * _N
    nq = nbt // _IC

    def _set_idx(qb, j, p):
      for i in range(_B // 16):
        idx_v[p, pl.ds(i * 16, 16)] = row_v[qb, j, pl.ds(i * 16, 16)] + cn16

    def _start(qb, j, p):
      if split:
        _set_idx(qb, j, p)
        return pltpu.async_copy(xs_hbm.at[idx_v.at[p]], rows_v.at[p],
                                sems[p])
      return pltpu.async_copy(xs_hbm.at[row_v.at[qb].at[j]], rows_v.at[p],
                              sems[p])

    pltpu.sync_copy(row_hbm.at[pl.ds(ibase, _IC)], row_v.at[0])
    pltpu.sync_copy(col_hbm.at[pl.ds(ibase, _IC)], col_v.at[0])

    def _pair(q2, carry):
      for qb in range(2):
        q = q2 * 2 + qb
        cbase = ibase + q * _IC

        def _absorb():
          pltpu.make_async_copy(row_hbm.at[pl.ds(cbase, _IC)],
                                row_v.at[qb], isem).wait()
          pltpu.make_async_copy(col_hbm.at[pl.ds(cbase, _IC)],
                                col_v.at[qb], isem).wait()
        if qb == 0:
          @pl.when(q2 > 0)
          def _():
            _absorb()
        else:
          _absorb()

        @pl.when(q + 1 < nq)
        def _():
          pltpu.async_copy(row_hbm.at[pl.ds(cbase + _IC, _IC)],
                           row_v.at[1 - qb], isem)
          pltpu.async_copy(col_hbm.at[pl.ds(cbase + _IC, _IC)],
                           col_v.at[1 - qb], isem)

        cv = col_v.at[qb]
        if gather:
          h = _start(qb, 0, 0)
          for j in range(_IC):
            hn = _start(qb, j + 1, (j + 1) % 2) if j + 1 < _IC else None
            h.wait()
            pltpu.sync_copy(rows_v.at[j % 2], acc.at[cv.at[j]], add=True)
            h = hn
        else:
          for j in range(_IC):
            pltpu.sync_copy(rows_v.at[1], acc.at[cv.at[j]], add=True)
      return carry

    lax.fori_loop(0, nq // 2, _pair, 0)
    plsc.subcore_barrier()

    src0 = _PAD_ROWS + s * och
    dst0 = c * _NP + s * och
    sizes = [_B] * (och // _B) + ([och % _B] if och % _B else [])
    offs = [_B * k for k in range(len(sizes))]
    nck = len(sizes)

    def _rd(k):
      return pltpu.async_copy(
          acc.at[pl.ds(src0 + offs[k], sizes[k])],
          rows_v.at[k % 2].at[pl.ds(0, sizes[k])], sems[k % 2])

    def _wr(k):
      return pltpu.async_copy(
          rows_v.at[k % 2].at[pl.ds(0, sizes[k])],
          out_hbm.at[pl.ds(dst0 + offs[k], sizes[k])], ssems[k % 2])

    hw = [None, None]
    hr = _rd(0)
    for k in range(nck):
      hrn = None
      if k + 1 < nck:
        if hw[(k + 1) % 2] is not None:
          hw[(k + 1) % 2].wait()
        hrn = _rd(k + 1)
      hr.wait()
      hw[k % 2] = _wr(k)
      hr = hrn
    for p in range(2):
      if hw[p] is not None:
        hw[p].wait()

  return prop


def _row_spec(d):
  return pl.BlockSpec((_BN, d), lambda i: (i, 0))


def _pair_spec(d):
  return pl.BlockSpec((2, _BN, d), lambda i: (0, i, 0))


def _full_spec(r, d):
  return pl.BlockSpec((r, d), lambda i: (0, 0))


def _tca(deg_ref, x_ref, xs_ref, dis_ref):
  dis = lax.rsqrt(deg_ref[...] + 1.0)
  xs_ref[...] = x_ref[...] * dis
  dis_ref[...] = dis


def _tcb(s1_ref, xs_ref, dis_ref, w_ref, b_ref, out_ref):
  s1 = s1_ref[...]
  dis = dis_ref[...]
  t = (s1[0] + s1[1] + xs_ref[...]) * dis
  pre = jnp.dot(t, w_ref[...], preferred_element_type=jnp.float32) + b_ref[...]
  h = jnp.maximum(pre, 0.0) * dis
  out_ref[0, :, :] = h[:, :128]
  out_ref[1, :, :] = h[:, 128:]


def _tcc(s2_ref, h1_ref, dis_ref, w2_ref, b2_ref, w3_ref, out_ref):
  s2 = s2_ref[...]
  h1 = h1_ref[...]
  dis = dis_ref[...]
  t = jnp.concatenate([s2[0] + h1[0], s2[1] + h1[1]], axis=1) * dis
  h2 = jnp.maximum(
      jnp.dot(t, w2_ref[...], preferred_element_type=jnp.float32)
      + b2_ref[...], 0.0)
  out_ref[...] = jnp.dot(h2, w3_ref[...],
                         preferred_element_type=jnp.float32) * dis


def _tcd(s3_ref, vs_ref, dis_ref, b3_ref, g_ref, lb_ref, out_ref):
  s3 = s3_ref[...]
  h = (s3[0] + s3[1] + vs_ref[...]) * dis_ref[...] + b3_ref[...]
  m = jnp.mean(h, axis=1, keepdims=True)
  v = jnp.mean((h - m) * (h - m), axis=1, keepdims=True)
  out_ref[...] = (h - m) * lax.rsqrt(v + 1e-5) * g_ref[...] + lb_ref[...]


def kernel(x, edge_index, W1, b1, W2, b2, W3, b3, ln_g, ln_b):
  n = x.shape[0]
  e = edge_index.shape[1]
  grid = (n // _BN,)
  epad = -(-e // (_NC * _NT * _B * 8)) * (_NC * _NT * _B * 8)
  nbt_half = epad // (_NC * _NT * _B)
  nbt_full = epad // (_NT * _B)
  p = epad - e

  pad_row = (jnp.arange(p, dtype=jnp.int32) * 37) % n
  pad_col = jnp.arange(p, dtype=jnp.int32) % _PAD_ROWS
  rowp = jnp.concatenate([edge_index[0], pad_row]).reshape(-1, _B)
  colp = jnp.concatenate([edge_index[1] + _PAD_ROWS, pad_col]).reshape(-1, _B)

  prop_part = _make_prop(nbt_half, "part")
  prop_splt = _make_prop(nbt_full, "split")
  prop_ones = _make_prop(nbt_half, "ones")

  degp = prop_ones(jnp.ones((8, _D), jnp.float32), rowp, colp)
  deg = degp[0:n, 0:1] + degp[_NP:_NP + n, 0:1]

  xs, dis = pl.pallas_call(
      _tca,
      grid=grid,
      in_specs=[_row_spec(1), _row_spec(128)],
      out_specs=[_row_spec(128), _row_spec(1)],
      out_shape=[
          jax.ShapeDtypeStruct((n, 128), jnp.float32),
          jax.ShapeDtypeStruct((n, 1), jnp.float32),
      ],
  )(deg, x)

  s1 = prop_part(xs, rowp, colp).reshape(2, _NP, _D)

  h1 = pl.pallas_call(
      _tcb,
      grid=grid,
      in_specs=[_pair_spec(128), _row_spec(128), _row_spec(1),
                _full_spec(128, 256), _full_spec(1, 256)],
      out_specs=_pair_spec(128),
      out_shape=jax.ShapeDtypeStruct((2, n, 128), jnp.float32),
  )(s1, xs, dis, W1, b1.reshape(1, -1))

  s2 = prop_splt(h1.reshape(2 * n, 128), rowp, colp).reshape(2, _NP, _D)

  vs = pl.pallas_call(
      _tcc,
      grid=grid,
      in_specs=[_pair_spec(128), _pair_spec(128), _row_spec(1),
                _full_spec(256, 256), _full_spec(1, 256),
                _full_spec(256, 128)],
      out_specs=_row_spec(128),
      out_shape=jax.ShapeDtypeStruct((n, 128), jnp.float32),
  )(s2, h1, dis, W2, b2.reshape(1, -1), W3)

  s3 = prop_part(vs, rowp, colp).reshape(2, _NP, _D)

  out = pl.pallas_call(
      _tcd,
      grid=grid,
      in_specs=[_pair_spec(128), _row_spec(128), _row_spec(1),
                _full_spec(1, 128), _full_spec(1, 128), _full_spec(1, 128)],
      out_specs=_row_spec(128),
      out_shape=jax.ShapeDtypeStruct((n, 128), jnp.float32),
  )(s3, vs, dis, b3.reshape(1, -1), ln_g.reshape(1, -1), ln_b.reshape(1, -1))

  return out


# speedup vs baseline: 23.8688x; 1.0187x over previous
import functools

import jax
import jax.numpy as jnp
from jax import lax
from jax.experimental import pallas as pl
from jax.experimental.pallas import tpu as pltpu
from jax.experimental.pallas import tpu_sc as plsc

_N = 10000
_NT = 16
_NC = 2
_B = 128
_D = 128
_PAD_ROWS = 16
_ACC_ROWS = 10240
_OCH = 632
_IC = 8
_NP = _NT * _OCH
_BN = 1000


def _make_prop(nbt, mode, ic=_IC):
  mesh = plsc.VectorSubcoreMesh(core_axis_name="c", subcore_axis_name="s")
  zch = _ACC_ROWS // _NT
  och = _OCH
  split = mode == "split"
  gather = mode != "ones"

  @functools.partial(
      pl.kernel,
      out_type=jax.ShapeDtypeStruct((_NC * _NP, _D), jnp.float32),
      mesh=mesh,
      scratch_types=[
          pltpu.VMEM((2, ic, _B), jnp.int32),
          pltpu.VMEM((2, ic, _B), jnp.int32),
          pltpu.VMEM((2, _B), jnp.int32),
          pltpu.VMEM((2, _B, _D), jnp.float32),
          pltpu.VMEM_SHARED((_ACC_ROWS, _D), jnp.float32),
          pltpu.SemaphoreType.DMA,
          pltpu.SemaphoreType.DMA,
          pltpu.SemaphoreType.DMA,
          pltpu.SemaphoreType.DMA,
          pltpu.SemaphoreType.DMA,
      ],
  )
  def prop(xs_hbm, row_hbm, col_hbm, out_hbm, row_v, col_v, idx_v, rows_v,
           acc, sem0, sem1, sem2, sem3, sem4):
    c = lax.axis_index("c")
    s = lax.axis_index("s")
    sems = (sem0, sem1)
    ssems = (sem2, sem3)
    isem = sem4

    b1val = jnp.full((16,), 0.0 if gather else 1.0, jnp.float32)
    zero16 = jnp.zeros((16,), jnp.float32)

    def _fill(r, carry):
      for i in range(_D // 16):
        rows_v[0, r, pl.ds(i * 16, 16)] = zero16
        rows_v[1, r, pl.ds(i * 16, 16)] = b1val
      return carry
    lax.fori_loop(0, _B, _fill, 0)
    zbase = s * zch
    hz = [pltpu.async_copy(rows_v.at[0], acc.at[pl.ds(zbase + k * _B, _B)],
                           sem2) for k in range(zch // _B)]
    for h in hz:
      h.wait()
    plsc.subcore_barrier()

    ibase = s * nbt if split else (c * _NT + s) * nbt
    cn16 = jnp.zeros((16,), jnp.int32) + c * _N
    nq = nbt // ic

    def _set_idx(qb, j, p):
      for i in range(_B // 16):
        idx_v[p, pl.ds(i * 16, 16)] = row_v[qb, j, pl.ds(i * 16, 16)] + cn16

    def _start(qb, j, p):
      if split:
        _set_idx(qb, j, p)
        return pltpu.async_copy(xs_hbm.at[idx_v.at[p]], rows_v.at[p],
                                sems[p])
      return pltpu.async_copy(xs_hbm.at[row_v.at[qb].at[j]], rows_v.at[p],
                              sems[p])

    pltpu.sync_copy(row_hbm.at[pl.ds(ibase, ic)], row_v.at[0])
    pltpu.sync_copy(col_hbm.at[pl.ds(ibase, ic)], col_v.at[0])

    def _pair(q2, carry):
      for qb in range(2):
        q = q2 * 2 + qb
        cbase = ibase + q * ic

        def _absorb():
          pltpu.make_async_copy(row_hbm.at[pl.ds(cbase, ic)],
                                row_v.at[qb], isem).wait()
          pltpu.make_async_copy(col_hbm.at[pl.ds(cbase, ic)],
                                col_v.at[qb], isem).wait()
        if qb == 0:
          @pl.when(q2 > 0)
          def _():
            _absorb()
        else:
          _absorb()

        @pl.when(q + 1 < nq)
        def _():
          pltpu.async_copy(row_hbm.at[pl.ds(cbase + ic, ic)],
                           row_v.at[1 - qb], isem)
          pltpu.async_copy(col_hbm.at[pl.ds(cbase + ic, ic)],
                           col_v.at[1 - qb], isem)

        cv = col_v.at[qb]
        if gather:
          h = _start(qb, 0, 0)
          for j in range(ic):
            hn = _start(qb, j + 1, (j + 1) % 2) if j + 1 < ic else None
            h.wait()
            pltpu.sync_copy(rows_v.at[j % 2], acc.at[cv.at[j]], add=True)
            h = hn
        else:
          for j in range(ic):
            pltpu.sync_copy(rows_v.at[1], acc.at[cv.at[j]], add=True)
      return carry

    lax.fori_loop(0, nq // 2, _pair, 0)
    plsc.subcore_barrier()

    src0 = _PAD_ROWS + s * och
    dst0 = c * _NP + s * och
    sizes = [_B] * (och // _B) + ([och % _B] if och % _B else [])
    offs = [_B * k for k in range(len(sizes))]
    nck = len(sizes)

    def _rd(k):
      return pltpu.async_copy(
          acc.at[pl.ds(src0 + offs[k], sizes[k])],
          rows_v.at[k % 2].at[pl.ds(0, sizes[k])], sems[k % 2])

    def _wr(k):
      return pltpu.async_copy(
          rows_v.at[k % 2].at[pl.ds(0, sizes[k])],
          out_hbm.at[pl.ds(dst0 + offs[k], sizes[k])], ssems[k % 2])

    hw = [None, None]
    hr = _rd(0)
    for k in range(nck):
      hrn = None
      if k + 1 < nck:
        if hw[(k + 1) % 2] is not None:
          hw[(k + 1) % 2].wait()
        hrn = _rd(k + 1)
      hr.wait()
      hw[k % 2] = _wr(k)
      hr = hrn
    for p in range(2):
      if hw[p] is not None:
        hw[p].wait()

  return prop


def _row_spec(d):
  return pl.BlockSpec((_BN, d), lambda i: (i, 0))


def _pair_spec(d):
  return pl.BlockSpec((2, _BN, d), lambda i: (0, i, 0))


def _full_spec(r, d):
  return pl.BlockSpec((r, d), lambda i: (0, 0))


def _tca(deg_ref, x_ref, xs_ref, dis_ref):
  dis = lax.rsqrt(deg_ref[...] + 1.0)
  xs_ref[...] = x_ref[...] * dis
  dis_ref[...] = dis


def _tcb(s1_ref, xs_ref, dis_ref, w_ref, b_ref, out_ref):
  s1 = s1_ref[...]
  dis = dis_ref[...]
  t = (s1[0] + s1[1] + xs_ref[...]) * dis
  pre = jnp.dot(t, w_ref[...], preferred_element_type=jnp.float32) + b_ref[...]
  h = jnp.maximum(pre, 0.0) * dis
  out_ref[0, :, :] = h[:, :128]
  out_ref[1, :, :] = h[:, 128:]


def _tcc(s2_ref, h1_ref, dis_ref, w2_ref, b2_ref, w3_ref, out_ref):
  s2 = s2_ref[...]
  h1 = h1_ref[...]
  dis = dis_ref[...]
  t = jnp.concatenate([s2[0] + h1[0], s2[1] + h1[1]], axis=1) * dis
  h2 = jnp.maximum(
      jnp.dot(t, w2_ref[...], preferred_element_type=jnp.float32)
      + b2_ref[...], 0.0)
  out_ref[...] = jnp.dot(h2, w3_ref[...],
                         preferred_element_type=jnp.float32) * dis


def _tcd(s3_ref, vs_ref, dis_ref, b3_ref, g_ref, lb_ref, out_ref):
  s3 = s3_ref[...]
  h = (s3[0] + s3[1] + vs_ref[...]) * dis_ref[...] + b3_ref[...]
  m = jnp.mean(h, axis=1, keepdims=True)
  v = jnp.mean((h - m) * (h - m), axis=1, keepdims=True)
  out_ref[...] = (h - m) * lax.rsqrt(v + 1e-5) * g_ref[...] + lb_ref[...]


def kernel(x, edge_index, W1, b1, W2, b2, W3, b3, ln_g, ln_b):
  n = x.shape[0]
  e = edge_index.shape[1]
  grid = (n // _BN,)
  epad = -(-e // (_NC * _NT * _B * 8)) * (_NC * _NT * _B * 8)
  nbt_half = epad // (_NC * _NT * _B)
  nbt_full = epad // (_NT * _B)
  p = epad - e

  pad_row = (jnp.arange(p, dtype=jnp.int32) * 37) % n
  pad_col = jnp.arange(p, dtype=jnp.int32) % _PAD_ROWS
  rowp = jnp.concatenate([edge_index[0], pad_row]).reshape(-1, _B)
  colp = jnp.concatenate([edge_index[1] + _PAD_ROWS, pad_col]).reshape(-1, _B)

  prop_part = _make_prop(nbt_half, "part")
  prop_splt = _make_prop(nbt_full, "split", ic=16)
  prop_ones = _make_prop(nbt_half, "ones")

  degp = prop_ones(jnp.ones((8, _D), jnp.float32), rowp, colp)
  deg = degp[0:n, 0:1] + degp[_NP:_NP + n, 0:1]

  xs, dis = pl.pallas_call(
      _tca,
      grid=grid,
      in_specs=[_row_spec(1), _row_spec(128)],
      out_specs=[_row_spec(128), _row_spec(1)],
      out_shape=[
          jax.ShapeDtypeStruct((n, 128), jnp.float32),
          jax.ShapeDtypeStruct((n, 1), jnp.float32),
      ],
  )(deg, x)

  s1 = prop_part(xs, rowp, colp).reshape(2, _NP, _D)

  h1 = pl.pallas_call(
      _tcb,
      grid=grid,
      in_specs=[_pair_spec(128), _row_spec(128), _row_spec(1),
                _full_spec(128, 256), _full_spec(1, 256)],
      out_specs=_pair_spec(128),
      out_shape=jax.ShapeDtypeStruct((2, n, 128), jnp.float32),
  )(s1, xs, dis, W1, b1.reshape(1, -1))

  s2 = prop_splt(h1.reshape(2 * n, 128), rowp, colp).reshape(2, _NP, _D)

  vs = pl.pallas_call(
      _tcc,
      grid=grid,
      in_specs=[_pair_spec(128), _pair_spec(128), _row_spec(1),
                _full_spec(256, 256), _full_spec(1, 256),
                _full_spec(256, 128)],
      out_specs=_row_spec(128),
      out_shape=jax.ShapeDtypeStruct((n, 128), jnp.float32),
  )(s2, h1, dis, W2, b2.reshape(1, -1), W3)

  s3 = prop_part(vs, rowp, colp).reshape(2, _NP, _D)

  out = pl.pallas_call(
      _tcd,
      grid=grid,
      in_specs=[_pair_spec(128), _row_spec(128), _row_spec(1),
                _full_spec(1, 128), _full_spec(1, 128), _full_spec(1, 128)],
      out_specs=_row_spec(128),
      out_shape=jax.ShapeDtypeStruct((n, 128), jnp.float32),
  )(s3, vs, dis, b3.reshape(1, -1), ln_g.reshape(1, -1), ln_b.reshape(1, -1))

  return out


# speedup vs baseline: 24.3965x; 1.0221x over previous
import functools

import jax
import jax.numpy as jnp
from jax import lax
from jax.experimental import pallas as pl
from jax.experimental.pallas import tpu as pltpu
from jax.experimental.pallas import tpu_sc as plsc

_N = 10000
_NT = 16
_NC = 2
_B = 128
_D = 128
_PAD_ROWS = 16
_ACC_ROWS = 10240
_OCH = 632
_IC = 8
_NP = _NT * _OCH
_BN = 1000


def _make_prop(nbt, mode, ic=_IC):
  mesh = plsc.VectorSubcoreMesh(core_axis_name="c", subcore_axis_name="s")
  zch = _ACC_ROWS // _NT
  och = _OCH
  split = mode == "split"
  gather = mode != "ones"

  @functools.partial(
      pl.kernel,
      out_type=jax.ShapeDtypeStruct((_NC * _NP, _D), jnp.float32),
      mesh=mesh,
      scratch_types=[
          pltpu.VMEM((2, ic, _B), jnp.int32),
          pltpu.VMEM((2, ic, _B), jnp.int32),
          pltpu.VMEM((2, _B), jnp.int32),
          pltpu.VMEM((2, _B, _D), jnp.float32),
          pltpu.VMEM_SHARED((_ACC_ROWS, _D), jnp.float32),
          pltpu.SemaphoreType.DMA,
          pltpu.SemaphoreType.DMA,
          pltpu.SemaphoreType.DMA,
          pltpu.SemaphoreType.DMA,
          pltpu.SemaphoreType.DMA,
      ],
  )
  def prop(xs_hbm, row_hbm, col_hbm, out_hbm, row_v, col_v, idx_v, rows_v,
           acc, sem0, sem1, sem2, sem3, sem4):
    c = lax.axis_index("c")
    s = lax.axis_index("s")
    sems = (sem0, sem1)
    ssems = (sem2, sem3)
    isem = sem4

    b1val = jnp.full((16,), 0.0 if gather else 1.0, jnp.float32)
    zero16 = jnp.zeros((16,), jnp.float32)

    def _fill(r, carry):
      for i in range(_D // 16):
        rows_v[0, r, pl.ds(i * 16, 16)] = zero16
        rows_v[1, r, pl.ds(i * 16, 16)] = b1val
      return carry
    lax.fori_loop(0, _B, _fill, 0)
    zbase = s * zch
    hz = [pltpu.async_copy(rows_v.at[0], acc.at[pl.ds(zbase + k * _B, _B)],
                           sem2) for k in range(zch // _B)]
    for h in hz:
      h.wait()
    plsc.subcore_barrier()

    ibase = s * nbt if split else (c * _NT + s) * nbt
    cn16 = jnp.zeros((16,), jnp.int32) + c * _N
    nq = nbt // ic

    def _set_idx(qb, j, p):
      for i in range(_B // 16):
        idx_v[p, pl.ds(i * 16, 16)] = row_v[qb, j, pl.ds(i * 16, 16)] + cn16

    def _start(qb, j, p):
      if split:
        _set_idx(qb, j, p)
        return pltpu.async_copy(xs_hbm.at[idx_v.at[p]], rows_v.at[p],
                                sems[p])
      return pltpu.async_copy(xs_hbm.at[row_v.at[qb].at[j]], rows_v.at[p],
                              sems[p])

    pltpu.sync_copy(row_hbm.at[pl.ds(ibase, ic)], row_v.at[0])
    pltpu.sync_copy(col_hbm.at[pl.ds(ibase, ic)], col_v.at[0])

    def _chunk(q, qb, absorb):
      cbase = ibase + q * ic

      def _absorb():
        pltpu.make_async_copy(row_hbm.at[pl.ds(cbase, ic)],
                              row_v.at[qb], isem).wait()
        pltpu.make_async_copy(col_hbm.at[pl.ds(cbase, ic)],
                              col_v.at[qb], isem).wait()
      if absorb is None:
        _absorb()
      elif absorb is not False:
        @pl.when(absorb)
        def _():
          _absorb()

      @pl.when(q + 1 < nq)
      def _():
        pltpu.async_copy(row_hbm.at[pl.ds(cbase + ic, ic)],
                         row_v.at[1 - qb], isem)
        pltpu.async_copy(col_hbm.at[pl.ds(cbase + ic, ic)],
                         col_v.at[1 - qb], isem)

      cv = col_v.at[qb]
      if gather:
        h = _start(qb, 0, 0)
        for j in range(ic):
          hn = _start(qb, j + 1, (j + 1) % 2) if j + 1 < ic else None
          h.wait()
          pltpu.sync_copy(rows_v.at[j % 2], acc.at[cv.at[j]], add=True)
          h = hn
      else:
        for j in range(ic):
          pltpu.sync_copy(rows_v.at[1], acc.at[cv.at[j]], add=True)

    def _pair(q2, carry):
      _chunk(q2 * 2, 0, q2 > 0)
      _chunk(q2 * 2 + 1, 1, None)
      return carry

    lax.fori_loop(0, nq // 2, _pair, 0)
    if nq % 2:
      _chunk(nq - 1, (nq - 1) % 2, False if nq == 1 else None)
    plsc.subcore_barrier()

    src0 = _PAD_ROWS + s * och
    dst0 = c * _NP + s * och
    sizes = [_B] * (och // _B) + ([och % _B] if och % _B else [])
    offs = [_B * k for k in range(len(sizes))]
    nck = len(sizes)

    def _rd(k):
      return pltpu.async_copy(
          acc.at[pl.ds(src0 + offs[k], sizes[k])],
          rows_v.at[k % 2].at[pl.ds(0, sizes[k])], sems[k % 2])

    def _wr(k):
      return pltpu.async_copy(
          rows_v.at[k % 2].at[pl.ds(0, sizes[k])],
          out_hbm.at[pl.ds(dst0 + offs[k], sizes[k])], ssems[k % 2])

    hw = [None, None]
    hr = _rd(0)
    for k in range(nck):
      hrn = None
      if k + 1 < nck:
        if hw[(k + 1) % 2] is not None:
          hw[(k + 1) % 2].wait()
        hrn = _rd(k + 1)
      hr.wait()
      hw[k % 2] = _wr(k)
      hr = hrn
    for p in range(2):
      if hw[p] is not None:
        hw[p].wait()

  return prop


def _row_spec(d):
  return pl.BlockSpec((_BN, d), lambda i: (i, 0))


def _pair_spec(d):
  return pl.BlockSpec((2, _BN, d), lambda i: (0, i, 0))


def _full_spec(r, d):
  return pl.BlockSpec((r, d), lambda i: (0, 0))


def _tca(deg_ref, x_ref, xs_ref, dis_ref):
  dis = lax.rsqrt(deg_ref[...] + 1.0)
  xs_ref[...] = x_ref[...] * dis
  dis_ref[...] = dis


def _tcb(s1_ref, xs_ref, dis_ref, w_ref, b_ref, out_ref):
  s1 = s1_ref[...]
  dis = dis_ref[...]
  t = (s1[0] + s1[1] + xs_ref[...]) * dis
  pre = jnp.dot(t, w_ref[...], preferred_element_type=jnp.float32) + b_ref[...]
  h = jnp.maximum(pre, 0.0) * dis
  out_ref[0, :, :] = h[:, :128]
  out_ref[1, :, :] = h[:, 128:]


def _tcc(s2_ref, h1_ref, dis_ref, w2_ref, b2_ref, w3_ref, out_ref):
  s2 = s2_ref[...]
  h1 = h1_ref[...]
  dis = dis_ref[...]
  t = jnp.concatenate([s2[0] + h1[0], s2[1] + h1[1]], axis=1) * dis
  h2 = jnp.maximum(
      jnp.dot(t, w2_ref[...], preferred_element_type=jnp.float32)
      + b2_ref[...], 0.0)
  out_ref[...] = jnp.dot(h2, w3_ref[...],
                         preferred_element_type=jnp.float32) * dis


def _tcd(s3_ref, vs_ref, dis_ref, b3_ref, g_ref, lb_ref, out_ref):
  s3 = s3_ref[...]
  h = (s3[0] + s3[1] + vs_ref[...]) * dis_ref[...] + b3_ref[...]
  m = jnp.mean(h, axis=1, keepdims=True)
  v = jnp.mean((h - m) * (h - m), axis=1, keepdims=True)
  out_ref[...] = (h - m) * lax.rsqrt(v + 1e-5) * g_ref[...] + lb_ref[...]


def kernel(x, edge_index, W1, b1, W2, b2, W3, b3, ln_g, ln_b):
  n = x.shape[0]
  e = edge_index.shape[1]
  grid = (n // _BN,)
  epad = -(-e // (_NC * _NT * _B * 8)) * (_NC * _NT * _B * 8)
  nbt_half = epad // (_NC * _NT * _B)
  nbt_full = epad // (_NT * _B)
  p = epad - e

  pad_row = (jnp.arange(p, dtype=jnp.int32) * 37) % n
  pad_col = jnp.arange(p, dtype=jnp.int32) % _PAD_ROWS
  rowp = jnp.concatenate([edge_index[0], pad_row]).reshape(-1, _B)
  colp = jnp.concatenate([edge_index[1] + _PAD_ROWS, pad_col]).reshape(-1, _B)

  prop_part = _make_prop(nbt_half, "part", ic=16)
  prop_splt = _make_prop(nbt_full, "split", ic=16)
  prop_ones = _make_prop(nbt_half, "ones", ic=16)

  degp = prop_ones(jnp.ones((8, _D), jnp.float32), rowp, colp)
  deg = degp[0:n, 0:1] + degp[_NP:_NP + n, 0:1]

  xs, dis = pl.pallas_call(
      _tca,
      grid=grid,
      in_specs=[_row_spec(1), _row_spec(128)],
      out_specs=[_row_spec(128), _row_spec(1)],
      out_shape=[
          jax.ShapeDtypeStruct((n, 128), jnp.float32),
          jax.ShapeDtypeStruct((n, 1), jnp.float32),
      ],
  )(deg, x)

  s1 = prop_part(xs, rowp, colp).reshape(2, _NP, _D)

  h1 = pl.pallas_call(
      _tcb,
      grid=grid,
      in_specs=[_pair_spec(128), _row_spec(128), _row_spec(1),
                _full_spec(128, 256), _full_spec(1, 256)],
      out_specs=_pair_spec(128),
      out_shape=jax.ShapeDtypeStruct((2, n, 128), jnp.float32),
  )(s1, xs, dis, W1, b1.reshape(1, -1))

  s2 = prop_splt(h1.reshape(2 * n, 128), rowp, colp).reshape(2, _NP, _D)

  vs = pl.pallas_call(
      _tcc,
      grid=grid,
      in_specs=[_pair_spec(128), _pair_spec(128), _row_spec(1),
                _full_spec(256, 256), _full_spec(1, 256),
                _full_spec(256, 128)],
      out_specs=_row_spec(128),
      out_shape=jax.ShapeDtypeStruct((n, 128), jnp.float32),
  )(s2, h1, dis, W2, b2.reshape(1, -1), W3)

  s3 = prop_part(vs, rowp, colp).reshape(2, _NP, _D)

  out = pl.pallas_call(
      _tcd,
      grid=grid,
      in_specs=[_pair_spec(128), _row_spec(128), _row_spec(1),
                _full_spec(1, 128), _full_spec(1, 128), _full_spec(1, 128)],
      out_specs=_row_spec(128),
      out_shape=jax.ShapeDtypeStruct((n, 128), jnp.float32),
  )(s3, vs, dis, b3.reshape(1, -1), ln_g.reshape(1, -1), ln_b.reshape(1, -1))

  return out
